# baseline (speedup 1.0000x reference)
import functools

import jax
import jax.numpy as jnp
from jax import lax
from jax.experimental import pallas as pl
from jax.experimental.pallas import tpu as pltpu
from jax.experimental.pallas import tpu_sc as plsc

_R = 512
_C = 512
_K = 16
_NSEG = 16
_F = 64
_FP = 128
_CH = 128
_INF = float("inf")
_BIGI = 2**30
_INTERPRET = False


def _enc_body(x_ref, w1_ref, b1_ref, w2_ref, b2_ref, h_ref):
    h = jnp.dot(x_ref[...], w1_ref[...], preferred_element_type=jnp.float32)
    h = jnp.maximum(h + b1_ref[...], 0.0)
    h = jnp.dot(h, w2_ref[...], preferred_element_type=jnp.float32)
    h_ref[...] = jnp.maximum(h + b2_ref[...], 0.0)


def _encoder(xp, w1, b1, w2, b2):
    npad, din = xp.shape
    return pl.pallas_call(
        _enc_body,
        grid=(npad // _R,),
        in_specs=[
            pl.BlockSpec((_R, din), lambda i: (i, 0)),
            pl.BlockSpec(w1.shape, lambda i: (0, 0)),
            pl.BlockSpec(b1.shape, lambda i: (0, 0)),
            pl.BlockSpec(w2.shape, lambda i: (0, 0)),
            pl.BlockSpec(b2.shape, lambda i: (0, 0)),
        ],
        out_specs=pl.BlockSpec((_R, _F), lambda i: (i, 0)),
        out_shape=jax.ShapeDtypeStruct((npad, _F), jnp.float32),
        interpret=_INTERPRET,
    )(xp, w1, b1, w2, b2)


def _knn_body(cb_lo_ref, cb_hi_ref, h_ref, bsub_ref, blane_ref, idx_ref,
              n2s_ref, n2l_ref):
    rb = pl.program_id(0)
    r0 = rb * _R

    @pl.when(rb == 0)
    def _():
        hsq = h_ref[...] * h_ref[...]
        n2s_ref[...] = jnp.sum(hsq, axis=1, keepdims=True)
        n2l_ref[...] = n2s_ref[...].T

    rows = h_ref[pl.ds(r0, _R), :]
    rn2 = n2l_ref[:, pl.ds(r0, _R)]
    blane = blane_ref[:, pl.ds(r0, _R)]

    def _dist_half(c0):
        cols = h_ref[pl.ds(c0, _C // 2), :]
        prod = lax.dot_general(cols, rows, (((1,), (1,)), ((), ())),
                               preferred_element_type=jnp.float32)
        cn2 = n2s_ref[pl.ds(c0, _C // 2), :]
        d = cn2 + rn2 - 2.0 * prod
        cid = c0 + lax.broadcasted_iota(jnp.int32, (_C // 2, _R), 0)
        rid = r0 + lax.broadcasted_iota(jnp.int32, (_C // 2, _R), 1)
        ok = (bsub_ref[pl.ds(c0, _C // 2), :] == blane) & (cid != rid)
        return jnp.where(ok, d, _INF), cid

    def col_step(j, carry):
        bv, bi = carry
        c0 = j * _C
        dtop, itop = _dist_half(c0)
        dbot, ibot = _dist_half(c0 + _C // 2)
        take = dtop <= dbot
        wv = jnp.minimum(dtop, dbot)
        wi = jnp.where(take, itop, ibot)
        pad_inf = jnp.full((_K, _R), _INF, jnp.float32)
        pad_id = jnp.full((_K, _R), _BIGI, jnp.int32)
        cv = jnp.concatenate([bv, wv], axis=0)
        ci = jnp.concatenate([bi, wi], axis=0)
        lv = jnp.concatenate([pad_inf, jnp.maximum(dtop, dbot)], axis=0)
        li = jnp.concatenate([pad_id, jnp.where(take, ibot, itop)], axis=0)
        nv, ni = [], []
        for _t in range(_K):
            m = jnp.min(cv, axis=0, keepdims=True)
            ismin = cv == m
            mi = jnp.min(jnp.where(ismin, ci, _BIGI), axis=0, keepdims=True)
            nv.append(m)
            ni.append(mi)
            kill = ismin & (ci == mi)
            cv = jnp.where(kill, lv, cv)
            ci = jnp.where(kill, li, ci)
            lv = jnp.where(kill, _INF, lv)
        return jnp.concatenate(nv, axis=0), jnp.concatenate(ni, axis=0)

    best_v = jnp.full((_K, _R), _INF, jnp.float32)
    best_i = lax.broadcasted_iota(jnp.int32, (_K, _R), 0)
    _, best_i = lax.fori_loop(cb_lo_ref[rb], cb_hi_ref[rb], col_step,
                              (best_v, best_i))
    idx_ref[...] = jnp.minimum(best_i, h_ref.shape[0] - 1)


def _knn(h, brow, bcol, cb_lo, cb_hi):
    npad = h.shape[0]
    nrb = npad // _R
    grid_spec = pltpu.PrefetchScalarGridSpec(
        num_scalar_prefetch=2,
        grid=(nrb,),
        in_specs=[
            pl.BlockSpec((npad, _F), lambda i, lo, hi: (0, 0)),
            pl.BlockSpec((npad, 1), lambda i, lo, hi: (0, 0)),
            pl.BlockSpec((1, npad), lambda i, lo, hi: (0, 0)),
        ],
        out_specs=pl.BlockSpec((_K, _R), lambda i, lo, hi: (0, i)),
        scratch_shapes=[pltpu.VMEM((npad, 1), jnp.float32),
                        pltpu.VMEM((1, npad), jnp.float32)],
    )
    return pl.pallas_call(
        _knn_body,
        grid_spec=grid_spec,
        out_shape=jax.ShapeDtypeStruct((_K, npad), jnp.int32),
        interpret=_INTERPRET,
    )(cb_lo, cb_hi, h, brow, bcol)


def _sc_gather(table, idxf):
    ng = idxf.shape[0]
    info = plsc.get_sparse_core_info()
    nw = info.num_cores * info.num_subcores
    per_w = ng // nw
    grp = 4 * _CH
    ngrp = per_w // grp
    mesh = plsc.VectorSubcoreMesh(core_axis_name="c", subcore_axis_name="s")

    @functools.partial(
        pl.kernel, mesh=mesh,
        out_type=jax.ShapeDtypeStruct((ng, _FP), jnp.float32),
        scratch_types=[
            pltpu.VMEM((per_w,), jnp.int32),
            pltpu.VMEM((grp, _FP), jnp.float32),
            pltpu.SemaphoreType.DMA,
        ],
    )
    def gk(table_hbm, idx_hbm, out_hbm, idx_v, rows_v, sem):
        wid = lax.axis_index("s") * info.num_cores + lax.axis_index("c")
        base = wid * per_w
        pltpu.sync_copy(idx_hbm.at[pl.ds(base, per_w)], idx_v)

        def step(t, carry):
            off = t * grp
            cps = [
                pltpu.async_copy(
                    table_hbm.at[idx_v.at[pl.ds(off + i * _CH, _CH)]],
                    rows_v.at[pl.ds(i * _CH, _CH)], sem)
                for i in range(grp // _CH)
            ]
            for cp in cps:
                cp.wait()
            pltpu.sync_copy(rows_v, out_hbm.at[pl.ds(base + off, grp)])
            return carry

        lax.fori_loop(0, ngrp, step, 0)

    return gk(table, idxf)


def _conv_body(gk_ref, h_ref, w1_ref, b1_ref, w2_ref, b2_ref, o_ref):
    k = pl.program_id(1)
    xi = h_ref[...]
    xj = gk_ref[0][:, :_F]
    feat = jnp.concatenate([xi, xj - xi], axis=1)
    t = jnp.maximum(jnp.dot(feat, w1_ref[...],
                            preferred_element_type=jnp.float32) + b1_ref[...],
                    0.0)
    h2 = jnp.dot(t, w2_ref[...], preferred_element_type=jnp.float32)

    @pl.when(k == 0)
    def _():
        o_ref[...] = h2

    @pl.when(k > 0)
    def _():
        o_ref[...] = jnp.maximum(o_ref[...], h2)

    @pl.when(k == _K - 1)
    def _():
        o_ref[...] = o_ref[...] + b2_ref[...]


def _conv(g3, h, w1, b1, w2, b2):
    npad = h.shape[0]
    nrb = npad // _R
    return pl.pallas_call(
        _conv_body,
        grid=(nrb, _K),
        in_specs=[
            pl.BlockSpec((1, _R, _FP), lambda i, k: (k, i, 0)),
            pl.BlockSpec((_R, _F), lambda i, k: (i, 0)),
            pl.BlockSpec((2 * _F, _F), lambda i, k: (0, 0)),
            pl.BlockSpec((1, _F), lambda i, k: (0, 0)),
            pl.BlockSpec((_F, _F), lambda i, k: (0, 0)),
            pl.BlockSpec((1, _F), lambda i, k: (0, 0)),
        ],
        out_specs=pl.BlockSpec((_R, _F), lambda i, k: (i, 0)),
        out_shape=jax.ShapeDtypeStruct((npad, _F), jnp.float32),
        interpret=_INTERPRET,
    )(g3, h, w1, b1, w2, b2)


def _pool_body(h_ref, brow_ref, ow1_ref, ob1_ref, ow2_ref, ob2_ref, o_ref,
               sum_ref, cnt_ref):
    rb = pl.program_id(0)
    nrb = pl.num_programs(0)

    @pl.when(rb == 0)
    def _():
        sum_ref[...] = jnp.zeros_like(sum_ref)
        cnt_ref[...] = jnp.zeros_like(cnt_ref)

    hblk = h_ref[...]
    bblk = brow_ref[...]
    for b in range(_NSEG):
        m = bblk == b
        sum_ref[pl.ds(b, 1), :] += jnp.sum(
            jnp.where(m, hblk, 0.0), axis=0, keepdims=True)
        cnt_ref[pl.ds(b, 1), :] += jnp.sum(
            jnp.where(m, 1.0, 0.0), axis=0, keepdims=True)

    @pl.when(rb == nrb - 1)
    def _():
        gmean = sum_ref[...] / jnp.maximum(cnt_ref[...], 1.0)
        o1 = jnp.dot(gmean, ow1_ref[...], preferred_element_type=jnp.float32)
        o1 = jnp.maximum(o1 + ob1_ref[...], 0.0)
        o_ref[...] = jnp.dot(o1, ow2_ref[...],
                             preferred_element_type=jnp.float32) + ob2_ref[...]


def _pool(h, brow, ow1, ob1, ow2, ob2):
    npad = h.shape[0]
    nrb = npad // _R
    return pl.pallas_call(
        _pool_body,
        grid=(nrb,),
        in_specs=[
            pl.BlockSpec((_R, _F), lambda i: (i, 0)),
            pl.BlockSpec((_R, 1), lambda i: (i, 0)),
            pl.BlockSpec((_F, 32), lambda i: (0, 0)),
            pl.BlockSpec((1, 32), lambda i: (0, 0)),
            pl.BlockSpec((32, 1), lambda i: (0, 0)),
            pl.BlockSpec((1, 1), lambda i: (0, 0)),
        ],
        out_specs=pl.BlockSpec((_NSEG, 1), lambda i: (0, 0)),
        out_shape=jax.ShapeDtypeStruct((_NSEG, 1), jnp.float32),
        scratch_shapes=[
            pltpu.VMEM((_NSEG, _F), jnp.float32),
            pltpu.VMEM((_NSEG, 1), jnp.float32),
        ],
        interpret=_INTERPRET,
    )(h, brow, ow1, ob1, ow2, ob2)


def kernel(x, pos, batch, enc_w1, enc_b1, enc_w2, enc_b2,
           c1_w1, c1_b1, c1_w2, c1_b2, c2_w1, c2_b1, c2_w2, c2_b2,
           out_w1, out_b1, out_w2, out_b2):
    n = x.shape[0]
    npad = ((n + _R - 1) // _R) * _R
    xp = jnp.pad(x, ((0, npad - n), (0, 0)))
    bp = jnp.pad(batch.astype(jnp.int32), (0, npad - n),
                 constant_values=_NSEG)
    brow = bp[:, None]
    bcol = bp[None, :]

    seg_start = jnp.searchsorted(
        bp, jnp.arange(_NSEG + 2, dtype=jnp.int32)).astype(jnp.int32)
    rb0 = jnp.arange(npad // _R, dtype=jnp.int32) * _R
    col_lo = seg_start[bp[rb0]]
    col_hi = seg_start[bp[rb0 + _R - 1] + 1]
    cb_lo = col_lo // _C
    cb_hi = (col_hi + _C - 1) // _C

    h = _encoder(xp, enc_w1, enc_b1.reshape(1, -1),
                 enc_w2, enc_b2.reshape(1, -1))

    def econv_round(hcur, w1, b1, w2, b2):
        idx = _knn(hcur, brow, bcol, cb_lo, cb_hi)
        idx_km = idx.reshape(-1)
        table = jnp.pad(hcur, ((0, 0), (0, _FP - _F)))
        gath = _sc_gather(table, idx_km)
        return _conv(gath.reshape(_K, npad, _FP), hcur, w1,
                     b1.reshape(1, -1), w2, b2.reshape(1, -1))

    h = econv_round(h, c1_w1, c1_b1, c1_w2, c1_b2)
    h = econv_round(h, c2_w1, c2_b1, c2_w2, c2_b2)

    o = _pool(h, brow, out_w1, out_b1.reshape(1, -1),
              out_w2, out_b2.reshape(1, -1))
    return o.reshape(-1)


# speedup vs baseline: 1.0333x; 1.0333x over previous
import functools

import jax
import jax.numpy as jnp
from jax import lax
from jax.experimental import pallas as pl
from jax.experimental.pallas import tpu as pltpu
from jax.experimental.pallas import tpu_sc as plsc

_R = 1024
_C = 512
_K = 16
_NSEG = 16
_F = 64
_FP = 128
_CH = 128
_INF = float("inf")
_BIGI = 2**30
_INTERPRET = False


def _enc_body(x_ref, w1_ref, b1_ref, w2_ref, b2_ref, h_ref):
    h = jnp.dot(x_ref[...], w1_ref[...], preferred_element_type=jnp.float32)
    h = jnp.maximum(h + b1_ref[...], 0.0)
    h = jnp.dot(h, w2_ref[...], preferred_element_type=jnp.float32)
    h_ref[...] = jnp.maximum(h + b2_ref[...], 0.0)


def _encoder(xp, w1, b1, w2, b2):
    npad, din = xp.shape
    return pl.pallas_call(
        _enc_body,
        grid=(npad // _R,),
        in_specs=[
            pl.BlockSpec((_R, din), lambda i: (i, 0)),
            pl.BlockSpec(w1.shape, lambda i: (0, 0)),
            pl.BlockSpec(b1.shape, lambda i: (0, 0)),
            pl.BlockSpec(w2.shape, lambda i: (0, 0)),
            pl.BlockSpec(b2.shape, lambda i: (0, 0)),
        ],
        out_specs=pl.BlockSpec((_R, _F), lambda i: (i, 0)),
        out_shape=jax.ShapeDtypeStruct((npad, _F), jnp.float32),
        interpret=_INTERPRET,
    )(xp, w1, b1, w2, b2)


def _knn_body(cb_lo_ref, cb_hi_ref, h_ref, bsub_ref, blane_ref, idx_ref,
              n2s_ref, n2l_ref):
    rb = pl.program_id(0)
    r0 = rb * _R

    @pl.when(rb == 0)
    def _():
        hsq = h_ref[...] * h_ref[...]
        n2s_ref[...] = jnp.sum(hsq, axis=1, keepdims=True)
        n2l_ref[...] = n2s_ref[...].T

    rows = h_ref[pl.ds(r0, _R), :]
    rn2 = n2l_ref[:, pl.ds(r0, _R)]
    blane = blane_ref[:, pl.ds(r0, _R)]

    def _dist_half(c0):
        cols = h_ref[pl.ds(c0, _C // 2), :]
        prod = lax.dot_general(cols, rows, (((1,), (1,)), ((), ())),
                               preferred_element_type=jnp.float32)
        cn2 = n2s_ref[pl.ds(c0, _C // 2), :]
        d = cn2 + rn2 - 2.0 * prod
        cid = c0 + lax.broadcasted_iota(jnp.int32, (_C // 2, _R), 0)
        rid = r0 + lax.broadcasted_iota(jnp.int32, (_C // 2, _R), 1)
        ok = (bsub_ref[pl.ds(c0, _C // 2), :] == blane) & (cid != rid)
        return jnp.where(ok, d, _INF), cid

    def col_step(j, carry):
        bv, bi = carry
        c0 = j * _C
        dtop, itop = _dist_half(c0)
        dbot, ibot = _dist_half(c0 + _C // 2)
        take = dtop <= dbot
        wv = jnp.minimum(dtop, dbot)
        wi = jnp.where(take, itop, ibot)
        pad_inf = jnp.full((_K, _R), _INF, jnp.float32)
        pad_id = jnp.full((_K, _R), _BIGI, jnp.int32)
        cv = jnp.concatenate([bv, wv], axis=0)
        ci = jnp.concatenate([bi, wi], axis=0)
        lv = jnp.concatenate([pad_inf, jnp.maximum(dtop, dbot)], axis=0)
        li = jnp.concatenate([pad_id, jnp.where(take, ibot, itop)], axis=0)
        nv, ni = [], []
        for _t in range(_K):
            m = jnp.min(cv, axis=0, keepdims=True)
            ismin = cv == m
            mi = jnp.min(jnp.where(ismin, ci, _BIGI), axis=0, keepdims=True)
            nv.append(m)
            ni.append(mi)
            kill = ismin & (ci == mi)
            cv = jnp.where(kill, lv, cv)
            ci = jnp.where(kill, li, ci)
            lv = jnp.where(kill, _INF, lv)
        return jnp.concatenate(nv, axis=0), jnp.concatenate(ni, axis=0)

    best_v = jnp.full((_K, _R), _INF, jnp.float32)
    best_i = lax.broadcasted_iota(jnp.int32, (_K, _R), 0)
    _, best_i = lax.fori_loop(cb_lo_ref[rb], cb_hi_ref[rb], col_step,
                              (best_v, best_i))
    idx_ref[...] = jnp.minimum(best_i, h_ref.shape[0] - 1)


def _knn(h, brow, bcol, cb_lo, cb_hi):
    npad = h.shape[0]
    nrb = npad // _R
    grid_spec = pltpu.PrefetchScalarGridSpec(
        num_scalar_prefetch=2,
        grid=(nrb,),
        in_specs=[
            pl.BlockSpec((npad, _F), lambda i, lo, hi: (0, 0)),
            pl.BlockSpec((npad, 1), lambda i, lo, hi: (0, 0)),
            pl.BlockSpec((1, npad), lambda i, lo, hi: (0, 0)),
        ],
        out_specs=pl.BlockSpec((_K, _R), lambda i, lo, hi: (0, i)),
        scratch_shapes=[pltpu.VMEM((npad, 1), jnp.float32),
                        pltpu.VMEM((1, npad), jnp.float32)],
    )
    return pl.pallas_call(
        _knn_body,
        grid_spec=grid_spec,
        out_shape=jax.ShapeDtypeStruct((_K, npad), jnp.int32),
        interpret=_INTERPRET,
    )(cb_lo, cb_hi, h, brow, bcol)


def _sc_gather(table, idxf):
    ng = idxf.shape[0]
    info = plsc.get_sparse_core_info()
    nw = info.num_cores * info.num_subcores
    per_w = ng // nw
    grp = 4 * _CH
    ngrp = per_w // grp
    mesh = plsc.VectorSubcoreMesh(core_axis_name="c", subcore_axis_name="s")

    @functools.partial(
        pl.kernel, mesh=mesh,
        out_type=jax.ShapeDtypeStruct((ng, _FP), jnp.float32),
        scratch_types=[
            pltpu.VMEM((per_w,), jnp.int32),
            pltpu.VMEM((grp, _FP), jnp.float32),
            pltpu.SemaphoreType.DMA,
        ],
    )
    def gk(table_hbm, idx_hbm, out_hbm, idx_v, rows_v, sem):
        wid = lax.axis_index("s") * info.num_cores + lax.axis_index("c")
        base = wid * per_w
        pltpu.sync_copy(idx_hbm.at[pl.ds(base, per_w)], idx_v)

        def step(t, carry):
            off = t * grp
            cps = [
                pltpu.async_copy(
                    table_hbm.at[idx_v.at[pl.ds(off + i * _CH, _CH)]],
                    rows_v.at[pl.ds(i * _CH, _CH)], sem)
                for i in range(grp // _CH)
            ]
            for cp in cps:
                cp.wait()
            pltpu.sync_copy(rows_v, out_hbm.at[pl.ds(base + off, grp)])
            return carry

        lax.fori_loop(0, ngrp, step, 0)

    return gk(table, idxf)


def _conv_body(gk_ref, h_ref, w1_ref, b1_ref, w2_ref, b2_ref, o_ref):
    k = pl.program_id(1)
    xi = h_ref[...]
    xj = gk_ref[0][:, :_F]
    feat = jnp.concatenate([xi, xj - xi], axis=1)
    t = jnp.maximum(jnp.dot(feat, w1_ref[...],
                            preferred_element_type=jnp.float32) + b1_ref[...],
                    0.0)
    h2 = jnp.dot(t, w2_ref[...], preferred_element_type=jnp.float32)

    @pl.when(k == 0)
    def _():
        o_ref[...] = h2

    @pl.when(k > 0)
    def _():
        o_ref[...] = jnp.maximum(o_ref[...], h2)

    @pl.when(k == _K - 1)
    def _():
        o_ref[...] = o_ref[...] + b2_ref[...]


def _conv(g3, h, w1, b1, w2, b2):
    npad = h.shape[0]
    nrb = npad // _R
    return pl.pallas_call(
        _conv_body,
        grid=(nrb, _K),
        in_specs=[
            pl.BlockSpec((1, _R, _FP), lambda i, k: (k, i, 0)),
            pl.BlockSpec((_R, _F), lambda i, k: (i, 0)),
            pl.BlockSpec((2 * _F, _F), lambda i, k: (0, 0)),
            pl.BlockSpec((1, _F), lambda i, k: (0, 0)),
            pl.BlockSpec((_F, _F), lambda i, k: (0, 0)),
            pl.BlockSpec((1, _F), lambda i, k: (0, 0)),
        ],
        out_specs=pl.BlockSpec((_R, _F), lambda i, k: (i, 0)),
        out_shape=jax.ShapeDtypeStruct((npad, _F), jnp.float32),
        interpret=_INTERPRET,
    )(g3, h, w1, b1, w2, b2)


def _pool_body(h_ref, brow_ref, ow1_ref, ob1_ref, ow2_ref, ob2_ref, o_ref,
               sum_ref, cnt_ref):
    rb = pl.program_id(0)
    nrb = pl.num_programs(0)

    @pl.when(rb == 0)
    def _():
        sum_ref[...] = jnp.zeros_like(sum_ref)
        cnt_ref[...] = jnp.zeros_like(cnt_ref)

    hblk = h_ref[...]
    bblk = brow_ref[...]
    for b in range(_NSEG):
        m = bblk == b
        sum_ref[pl.ds(b, 1), :] += jnp.sum(
            jnp.where(m, hblk, 0.0), axis=0, keepdims=True)
        cnt_ref[pl.ds(b, 1), :] += jnp.sum(
            jnp.where(m, 1.0, 0.0), axis=0, keepdims=True)

    @pl.when(rb == nrb - 1)
    def _():
        gmean = sum_ref[...] / jnp.maximum(cnt_ref[...], 1.0)
        o1 = jnp.dot(gmean, ow1_ref[...], preferred_element_type=jnp.float32)
        o1 = jnp.maximum(o1 + ob1_ref[...], 0.0)
        o_ref[...] = jnp.dot(o1, ow2_ref[...],
                             preferred_element_type=jnp.float32) + ob2_ref[...]


def _pool(h, brow, ow1, ob1, ow2, ob2):
    npad = h.shape[0]
    nrb = npad // _R
    return pl.pallas_call(
        _pool_body,
        grid=(nrb,),
        in_specs=[
            pl.BlockSpec((_R, _F), lambda i: (i, 0)),
            pl.BlockSpec((_R, 1), lambda i: (i, 0)),
            pl.BlockSpec((_F, 32), lambda i: (0, 0)),
            pl.BlockSpec((1, 32), lambda i: (0, 0)),
            pl.BlockSpec((32, 1), lambda i: (0, 0)),
            pl.BlockSpec((1, 1), lambda i: (0, 0)),
        ],
        out_specs=pl.BlockSpec((_NSEG, 1), lambda i: (0, 0)),
        out_shape=jax.ShapeDtypeStruct((_NSEG, 1), jnp.float32),
        scratch_shapes=[
            pltpu.VMEM((_NSEG, _F), jnp.float32),
            pltpu.VMEM((_NSEG, 1), jnp.float32),
        ],
        interpret=_INTERPRET,
    )(h, brow, ow1, ob1, ow2, ob2)


def kernel(x, pos, batch, enc_w1, enc_b1, enc_w2, enc_b2,
           c1_w1, c1_b1, c1_w2, c1_b2, c2_w1, c2_b1, c2_w2, c2_b2,
           out_w1, out_b1, out_w2, out_b2):
    n = x.shape[0]
    npad = ((n + _R - 1) // _R) * _R
    xp = jnp.pad(x, ((0, npad - n), (0, 0)))
    bp = jnp.pad(batch.astype(jnp.int32), (0, npad - n),
                 constant_values=_NSEG)
    brow = bp[:, None]
    bcol = bp[None, :]

    seg_start = jnp.searchsorted(
        bp, jnp.arange(_NSEG + 2, dtype=jnp.int32)).astype(jnp.int32)
    rb0 = jnp.arange(npad // _R, dtype=jnp.int32) * _R
    col_lo = seg_start[bp[rb0]]
    col_hi = seg_start[bp[rb0 + _R - 1] + 1]
    cb_lo = col_lo // _C
    cb_hi = (col_hi + _C - 1) // _C

    h = _encoder(xp, enc_w1, enc_b1.reshape(1, -1),
                 enc_w2, enc_b2.reshape(1, -1))

    def econv_round(hcur, w1, b1, w2, b2):
        idx = _knn(hcur, brow, bcol, cb_lo, cb_hi)
        idx_km = idx.reshape(-1)
        table = jnp.pad(hcur, ((0, 0), (0, _FP - _F)))
        gath = _sc_gather(table, idx_km)
        return _conv(gath.reshape(_K, npad, _FP), hcur, w1,
                     b1.reshape(1, -1), w2, b2.reshape(1, -1))

    h = econv_round(h, c1_w1, c1_b1, c1_w2, c1_b2)
    h = econv_round(h, c2_w1, c2_b1, c2_w2, c2_b2)

    o = _pool(h, brow, out_w1, out_b1.reshape(1, -1),
              out_w2, out_b2.reshape(1, -1))
    return o.reshape(-1)
